# baseline (speedup 1.0000x reference)
import functools

import jax
import jax.numpy as jnp
from jax import lax
from jax.experimental import pallas as pl
from jax.experimental.pallas import tpu as pltpu
from jax.experimental.pallas import tpu_sc as plsc

MAXV = 100000
NCH = 26
DIM = 16
LANES = 16
CHUNK = 128
NW = 32
NBUF = 4
AHEAD = 2


def _bow_kernel(B):
    nb = B // NW
    nsub = nb // CHUNK

    mesh = plsc.VectorSubcoreMesh(core_axis_name="c", subcore_axis_name="s")

    @functools.partial(
        pl.kernel,
        mesh=mesh,
        out_type=jax.ShapeDtypeStruct((B, NCH * DIM), jnp.float32),
        compiler_params=pltpu.CompilerParams(use_tc_tiling_on_sc=False),
        scratch_types=[
            pltpu.VMEM((NCH * nsub, CHUNK), jnp.int32),
            pltpu.VMEM((NBUF, nb, DIM), jnp.float32),
            pltpu.SemaphoreType.DMA,
            pltpu.SemaphoreType.DMA,
            pltpu.SemaphoreType.DMA,
        ],
    )
    def k(idx_hbm, table_hbm, out_hbm, idx_v, rows_v, isem, gsem, wsem):
        wid = lax.axis_index("s") * 2 + lax.axis_index("c")
        base = wid * nb

        for ch in range(NCH):
            pltpu.async_copy(
                idx_hbm.at[ch, pl.ds(wid * nsub, nsub)],
                idx_v.at[pl.ds(ch * nsub, nsub)],
                isem,
            )
        for ch in range(NCH):
            pltpu.make_async_copy(
                idx_hbm.at[ch, pl.ds(wid * nsub, nsub)],
                idx_v.at[pl.ds(ch * nsub, nsub)],
                isem,
            ).wait()

        def add_offsets(ch):
            off = ch * MAXV
            for j in range(nsub):
                for g in range(CHUNK // LANES):
                    sl = pl.ds(g * LANES, LANES)
                    idx_v[ch * nsub + j, sl] = idx_v[ch * nsub + j, sl] + off

        def fire_gathers(ch, buf):
            for j in range(nsub):
                pltpu.async_copy(
                    table_hbm.at[idx_v.at[ch * nsub + j]],
                    rows_v.at[buf, pl.ds(j * CHUNK, CHUNK)],
                    gsem,
                )

        def wait_gathers(ch, buf):
            for j in range(nsub):
                pltpu.make_async_copy(
                    table_hbm.at[idx_v.at[ch * nsub + j]],
                    rows_v.at[buf, pl.ds(j * CHUNK, CHUNK)],
                    gsem,
                ).wait()

        def out_slice(ch):
            return out_hbm.at[pl.ds(base, nb), pl.ds(ch * DIM, DIM)]

        for p in range(AHEAD):
            add_offsets(p)
            fire_gathers(p, p % NBUF)

        def body(c, _):
            buf = lax.rem(c, NBUF)

            @pl.when(c >= AHEAD)
            def _():
                pltpu.make_async_copy(
                    rows_v.at[lax.rem(c - AHEAD, NBUF)],
                    out_slice(c - AHEAD),
                    wsem,
                ).wait()

            @pl.when(c + AHEAD < NCH)
            def _():
                nxt = c + AHEAD
                off = nxt * MAXV
                for j in range(nsub):
                    for g in range(CHUNK // LANES):
                        sl = pl.ds(g * LANES, LANES)
                        row = nxt * nsub + j
                        idx_v[row, sl] = idx_v[row, sl] + off
                for j in range(nsub):
                    pltpu.async_copy(
                        table_hbm.at[idx_v.at[nxt * nsub + j]],
                        rows_v.at[lax.rem(nxt, NBUF), pl.ds(j * CHUNK, CHUNK)],
                        gsem,
                    )

            wait_gathers(c, buf)
            pltpu.async_copy(rows_v.at[buf], out_slice(c), wsem)
            return 0

        lax.fori_loop(0, NCH, body, 0)

        for c in range(NCH - AHEAD, NCH):
            pltpu.make_async_copy(
                rows_v.at[c % NBUF], out_slice(c), wsem
            ).wait()

    return k


def kernel(inputs, table):
    orig_shape = inputs.shape
    flat = inputs.reshape(-1, orig_shape[-1])
    B = flat.shape[0]
    idx_t = flat.astype(jnp.int32).T.reshape(NCH, B // CHUNK, CHUNK)
    out = _bow_kernel(B)(idx_t, table)
    return out.reshape(orig_shape[:-1] + (NCH * DIM,))
